# Initial kernel scaffold; baseline (speedup 1.0000x reference)
import jax
import jax.numpy as jnp
from jax.experimental import pallas as pl


def kernel(generated, target):
    raise NotImplementedError("write your pallas kernel here")


# speedup vs baseline: 29.4457x; 29.4457x over previous
import functools

import jax
import jax.numpy as jnp
from jax import lax
from jax.experimental import pallas as pl
from jax.experimental.pallas import tpu as pltpu
from jax.experimental.pallas import tpu_sc as plsc

_BINS = 64
_EPS = 1e-08
_NC, _NS, _L = 2, 16, 16
_NW = _NC * _NS
_S = 512 * 512
_P = 32768
_PIECES = _S // _P
_U = 8
_ACC = 4 * _BINS * _L

def _sc_hist_body(gen_hbm, tgt_hbm, out_hbm, buf0, acc):
    w = lax.axis_index("s") * _NC + lax.axis_index("c")

    zeros = jnp.zeros((_L,), jnp.float32)

    def zero_body(i, _):
        acc[pl.ds(i * _L, _L)] = zeros
        return 0

    lax.fori_loop(0, _ACC // _L, zero_body, 0)

    lane = lax.iota(jnp.int32, _L)
    ones = jnp.ones((_L,), jnp.float32)

    for h, src in ((0, gen_hbm), (1, gen_hbm), (2, tgt_hbm), (3, tgt_hbm)):
        ch = 1 + (h & 1)
        base = (3 * w + ch) * _S
        off_vec = lane + h * _BINS * _L

        def piece_body(p, _, src=src, base=base, off_vec=off_vec):
            pltpu.sync_copy(src.at[pl.ds(base + p * _P, _P)], buf0)

            def vec_body(i, _, off_vec=off_vec):
                for u in range(_U):
                    v = buf0[pl.ds((i * _U + u) * _L, _L)]
                    t = (v + 1.0) * 0.5
                    t = jnp.minimum(jnp.maximum(t, 0.0), 1.0) * 64.0
                    b = jnp.minimum(t.astype(jnp.int32), _BINS - 1)
                    plsc.addupdate_scatter(acc, [b * _L + off_vec], ones)
                return 0

            lax.fori_loop(0, _P // _L // _U, vec_body, 0)
            return 0

        lax.fori_loop(0, _PIECES, piece_body, 0)

    pltpu.sync_copy(acc, out_hbm.at[w])


def _finalize_body(x_ref, o_ref):
    x = x_ref[...]
    s = jnp.sum(x, axis=2)
    s = jnp.sum(s, axis=0, keepdims=True)

    def _norm(hh):
        hh = hh + _EPS
        return hh / jnp.sum(hh)

    g1 = _norm(s[:, 0:_BINS])
    g2 = _norm(s[:, _BINS:2 * _BINS])
    t1 = _norm(s[:, 2 * _BINS:3 * _BINS])
    t2 = _norm(s[:, 3 * _BINS:4 * _BINS])
    kl = (jnp.sum(t1 * (jnp.log(t1) - jnp.log(g1 + _EPS)))
          + jnp.sum(t2 * (jnp.log(t2) - jnp.log(g2 + _EPS))))
    o_ref[0, 0] = kl / (2 * _BINS)


@functools.lru_cache(maxsize=None)
def _build_sc_hist():
    mesh = plsc.VectorSubcoreMesh(
        core_axis_name="c", subcore_axis_name="s",
        num_cores=_NC, num_subcores=_NS)
    return pl.kernel(
        _sc_hist_body,
        out_type=jax.ShapeDtypeStruct((_NW, _ACC), jnp.float32),
        mesh=mesh,
        scratch_types=[
            pltpu.VMEM((_P,), jnp.float32),
            pltpu.VMEM((_ACC,), jnp.float32),
        ],
        compiler_params=pltpu.CompilerParams(needs_layout_passes=False),
    )


def kernel(generated, target):
    parts = _build_sc_hist()(generated.reshape(-1), target.reshape(-1))
    parts3 = parts.reshape(_NW, 4 * _BINS, _L)
    out = pl.pallas_call(
        _finalize_body,
        out_shape=jax.ShapeDtypeStruct((1, 1), jnp.float32),
        out_specs=pl.BlockSpec(memory_space=pltpu.SMEM),
    )(parts3)
    return out.reshape(())


# speedup vs baseline: 32.7830x; 1.1133x over previous
import functools

import jax
import jax.numpy as jnp
from jax import lax
from jax.experimental import pallas as pl
from jax.experimental.pallas import tpu as pltpu
from jax.experimental.pallas import tpu_sc as plsc

_BINS = 64
_EPS = 1e-08
_NC, _NS, _L = 2, 16, 16
_NW = _NC * _NS
_W = 512
_R = 64
_PIECES = 512 // _R
_U = 8
_ACC = 4 * _BINS * _L


def _sc_hist_body(gen_hbm, tgt_hbm, out_hbm, buf0, acc):
    w = lax.axis_index("s") * _NC + lax.axis_index("c")

    zeros = jnp.zeros((_L,), jnp.float32)

    def zero_body(i, _):
        acc[pl.ds(i * _L, _L)] = zeros
        return 0

    lax.fori_loop(0, _ACC // _L, zero_body, 0)

    lane = lax.iota(jnp.int32, _L)
    ones = jnp.ones((_L,), jnp.float32)

    for h, src in ((0, gen_hbm), (1, gen_hbm), (2, tgt_hbm), (3, tgt_hbm)):
        ch = 1 + (h & 1)
        off_vec = lane + h * _BINS * _L

        def piece_body(p, _, src=src, off_vec=off_vec):
            pltpu.sync_copy(src.at[w, ch, pl.ds(p * _R, _R), :], buf0)

            def row_body(i, _, off_vec=off_vec):
                def col_body(j, _, i=i, off_vec=off_vec):
                    for u in range(_U):
                        v = buf0[i, pl.ds((j * _U + u) * _L, _L)]
                        t = (v + 1.0) * 0.5
                        t = jnp.minimum(jnp.maximum(t, 0.0), 1.0) * 64.0
                        b = jnp.minimum(t.astype(jnp.int32), _BINS - 1)
                        plsc.addupdate_scatter(acc, [b * _L + off_vec], ones)
                    return 0

                lax.fori_loop(0, _W // _L // _U, col_body, 0)
                return 0

            lax.fori_loop(0, _R, row_body, 0)
            return 0

        lax.fori_loop(0, _PIECES, piece_body, 0)

    pltpu.sync_copy(acc, out_hbm.at[w])


@functools.lru_cache(maxsize=None)
def _build_sc_hist():
    mesh = plsc.VectorSubcoreMesh(
        core_axis_name="c", subcore_axis_name="s",
        num_cores=_NC, num_subcores=_NS)
    return pl.kernel(
        _sc_hist_body,
        out_type=jax.ShapeDtypeStruct((_NW, _ACC), jnp.float32),
        mesh=mesh,
        scratch_types=[
            pltpu.VMEM((_R, _W), jnp.float32),
            pltpu.VMEM((_ACC,), jnp.float32),
        ],
        compiler_params=pltpu.CompilerParams(needs_layout_passes=False),
    )


def _finalize_body(x_ref, o_ref):
    x = x_ref[...]
    s = jnp.sum(x, axis=2)
    s = jnp.sum(s, axis=0, keepdims=True)

    def _norm(hh):
        hh = hh + _EPS
        return hh / jnp.sum(hh)

    g1 = _norm(s[:, 0:_BINS])
    g2 = _norm(s[:, _BINS:2 * _BINS])
    t1 = _norm(s[:, 2 * _BINS:3 * _BINS])
    t2 = _norm(s[:, 3 * _BINS:4 * _BINS])
    kl = (jnp.sum(t1 * (jnp.log(t1) - jnp.log(g1 + _EPS)))
          + jnp.sum(t2 * (jnp.log(t2) - jnp.log(g2 + _EPS))))
    o_ref[0, 0] = kl / (2 * _BINS)


def kernel(generated, target):
    parts = _build_sc_hist()(generated, target)
    parts3 = parts.reshape(_NW, 4 * _BINS, _L)
    out = pl.pallas_call(
        _finalize_body,
        out_shape=jax.ShapeDtypeStruct((1, 1), jnp.float32),
        out_specs=pl.BlockSpec(memory_space=pltpu.SMEM),
    )(parts3)
    return out.reshape(())


# speedup vs baseline: 149.1242x; 4.5488x over previous
import functools

import jax
import jax.numpy as jnp
from jax import lax
from jax.experimental import pallas as pl
from jax.experimental.pallas import tpu as pltpu
from jax.experimental.pallas import tpu_sc as plsc

_BINS = 64
_EPS = 1e-08
_NC, _NS, _L = 2, 16, 16
_NW = _NC * _NS
_W = 512
_R = 64
_PIECES = 512 // _R
_U = 8
_ACC = 4 * _BINS * _L


def _sc_hist_body(gen_hbm, tgt_hbm, out_hbm, buf0, acc):
    w = lax.axis_index("s") * _NC + lax.axis_index("c")

    zeros = jnp.zeros((_L,), jnp.float32)

    def zero_body(i, _):
        acc[pl.ds(i * _L, _L)] = zeros
        return 0

    lax.fori_loop(0, _ACC // _L, zero_body, 0)

    lane = lax.iota(jnp.int32, _L)
    ones = jnp.ones((_L,), jnp.float32)

    for h, src in ((0, gen_hbm), (1, gen_hbm), (2, tgt_hbm), (3, tgt_hbm)):
        ch = 1 + (h & 1)
        off_vec = lane + h * _BINS * _L

        def piece_body(p, _, src=src, off_vec=off_vec):
            pltpu.sync_copy(src.at[w, ch, pl.ds(p * _R, _R), :], buf0)

            @plsc.parallel_loop(0, _R * _W // _L, unroll=_U)
            def _(vi, off_vec=off_vec):
                r = vi >> 5
                c = (vi & 31) * _L
                v = buf0[r, pl.ds(c, _L)]
                t = (v + 1.0) * 0.5
                t = jnp.minimum(jnp.maximum(t, 0.0), 1.0) * 64.0
                b = jnp.minimum(t.astype(jnp.int32), _BINS - 1)
                plsc.addupdate_scatter(acc, [b * _L + off_vec], ones)

            return 0

        lax.fori_loop(0, _PIECES, piece_body, 0)

    pltpu.sync_copy(acc, out_hbm.at[w])


@functools.lru_cache(maxsize=None)
def _build_sc_hist():
    mesh = plsc.VectorSubcoreMesh(
        core_axis_name="c", subcore_axis_name="s",
        num_cores=_NC, num_subcores=_NS)
    return pl.kernel(
        _sc_hist_body,
        out_type=jax.ShapeDtypeStruct((_NW, _ACC), jnp.float32),
        mesh=mesh,
        scratch_types=[
            pltpu.VMEM((_R, _W), jnp.float32),
            pltpu.VMEM((_ACC,), jnp.float32),
        ],
        compiler_params=pltpu.CompilerParams(needs_layout_passes=False),
    )


def _finalize_body(x_ref, o_ref):
    x = x_ref[...]
    s = jnp.sum(x, axis=2)
    s = jnp.sum(s, axis=0, keepdims=True)

    def _norm(hh):
        hh = hh + _EPS
        return hh / jnp.sum(hh)

    g1 = _norm(s[:, 0:_BINS])
    g2 = _norm(s[:, _BINS:2 * _BINS])
    t1 = _norm(s[:, 2 * _BINS:3 * _BINS])
    t2 = _norm(s[:, 3 * _BINS:4 * _BINS])
    kl = (jnp.sum(t1 * (jnp.log(t1) - jnp.log(g1 + _EPS)))
          + jnp.sum(t2 * (jnp.log(t2) - jnp.log(g2 + _EPS))))
    o_ref[0, 0] = kl / (2 * _BINS)


def kernel(generated, target):
    parts = _build_sc_hist()(generated, target)
    parts3 = parts.reshape(_NW, 4 * _BINS, _L)
    out = pl.pallas_call(
        _finalize_body,
        out_shape=jax.ShapeDtypeStruct((1, 1), jnp.float32),
        out_specs=pl.BlockSpec(memory_space=pltpu.SMEM),
    )(parts3)
    return out.reshape(())


# speedup vs baseline: 185.2729x; 1.2424x over previous
import functools

import jax
import jax.numpy as jnp
from jax import lax
from jax.experimental import pallas as pl
from jax.experimental.pallas import tpu as pltpu
from jax.experimental.pallas import tpu_sc as plsc

_BINS = 64
_EPS = 1e-08
_NC, _NS, _L = 2, 16, 16
_NW = _NC * _NS
_W = 512
_R = 64
_PIECES = 512 // _R
_U = 8
_ACC = 4 * _BINS * _L


def _sc_hist_body(gen_hbm, tgt_hbm, out_hbm, buf0, buf1, acc, sem0, sem1):
    w = lax.axis_index("s") * _NC + lax.axis_index("c")

    zeros = jnp.zeros((_L,), jnp.float32)

    def zero_body(i, _):
        acc[pl.ds(i * _L, _L)] = zeros
        return 0

    lax.fori_loop(0, _ACC // _L, zero_body, 0)

    lane = lax.iota(jnp.int32, _L)
    ones = jnp.ones((_L,), jnp.float32)

    def compute_piece(buf, off_vec):
        @plsc.parallel_loop(0, _R * _W // _L, unroll=_U)
        def _(vi):
            r = vi >> 5
            c = (vi & 31) * _L
            v = buf[r, pl.ds(c, _L)]
            t = (v + 1.0) * 0.5
            t = jnp.minimum(jnp.maximum(t, 0.0), 1.0) * 64.0
            b = jnp.minimum(t.astype(jnp.int32), _BINS - 1)
            plsc.addupdate_scatter(acc, [b * _L + off_vec], ones)

    for h, src in ((0, gen_hbm), (1, gen_hbm), (2, tgt_hbm), (3, tgt_hbm)):
        ch = 1 + (h & 1)
        off_vec = lane + h * _BINS * _L

        def piece_src(p, src=src, ch=ch):
            return src.at[w, ch, pl.ds(p * _R, _R), :]

        pltpu.async_copy(piece_src(0), buf0, sem0)

        def pair_body(j, _, piece_src=piece_src, off_vec=off_vec):
            pltpu.async_copy(piece_src(2 * j + 1), buf1, sem1)
            pltpu.make_async_copy(piece_src(0), buf0, sem0).wait()
            compute_piece(buf0, off_vec)

            @pl.when(j < _PIECES // 2 - 1)
            def _():
                pltpu.async_copy(piece_src(2 * j + 2), buf0, sem0)

            pltpu.make_async_copy(piece_src(0), buf1, sem1).wait()
            compute_piece(buf1, off_vec)
            return 0

        lax.fori_loop(0, _PIECES // 2, pair_body, 0)

    pltpu.sync_copy(acc, out_hbm.at[w])


@functools.lru_cache(maxsize=None)
def _build_sc_hist():
    mesh = plsc.VectorSubcoreMesh(
        core_axis_name="c", subcore_axis_name="s",
        num_cores=_NC, num_subcores=_NS)
    return pl.kernel(
        _sc_hist_body,
        out_type=jax.ShapeDtypeStruct((_NW, _ACC), jnp.float32),
        mesh=mesh,
        scratch_types=[
            pltpu.VMEM((_R, _W), jnp.float32),
            pltpu.VMEM((_R, _W), jnp.float32),
            pltpu.VMEM((_ACC,), jnp.float32),
            pltpu.SemaphoreType.DMA,
            pltpu.SemaphoreType.DMA,
        ],
        compiler_params=pltpu.CompilerParams(needs_layout_passes=False),
    )


def _finalize_body(x_ref, o_ref):
    x = x_ref[...]
    s = jnp.sum(x, axis=2)
    s = jnp.sum(s, axis=0, keepdims=True)

    def _norm(hh):
        hh = hh + _EPS
        return hh / jnp.sum(hh)

    g1 = _norm(s[:, 0:_BINS])
    g2 = _norm(s[:, _BINS:2 * _BINS])
    t1 = _norm(s[:, 2 * _BINS:3 * _BINS])
    t2 = _norm(s[:, 3 * _BINS:4 * _BINS])
    kl = (jnp.sum(t1 * (jnp.log(t1) - jnp.log(g1 + _EPS)))
          + jnp.sum(t2 * (jnp.log(t2) - jnp.log(g2 + _EPS))))
    o_ref[0, 0] = kl / (2 * _BINS)


def kernel(generated, target):
    parts = _build_sc_hist()(generated, target)
    parts3 = parts.reshape(_NW, 4 * _BINS, _L)
    out = pl.pallas_call(
        _finalize_body,
        out_shape=jax.ShapeDtypeStruct((1, 1), jnp.float32),
        out_specs=pl.BlockSpec(memory_space=pltpu.SMEM),
    )(parts3)
    return out.reshape(())


# speedup vs baseline: 236.0546x; 1.2741x over previous
import functools

import jax
import jax.numpy as jnp
from jax import lax
from jax.experimental import pallas as pl
from jax.experimental.pallas import tpu as pltpu
from jax.experimental.pallas import tpu_sc as plsc

_BINS = 64
_EPS = 1e-08
_NC, _NS, _L = 2, 16, 16
_NW = _NC * _NS
_W = 512
_R = 64
_PIECES = 512 // _R
_U = 8
_ACC = 4 * _BINS * _L


def _sc_hist_body(gen_hbm, tgt_hbm, out_hbm, buf0, buf1, acc, sem0, sem1):
    w = lax.axis_index("s") * _NC + lax.axis_index("c")

    zeros = jnp.zeros((_L,), jnp.float32)

    def zero_body(i, _):
        acc[pl.ds(i * _L, _L)] = zeros
        return 0

    lax.fori_loop(0, _ACC // _L, zero_body, 0)

    lane = lax.iota(jnp.int32, _L)
    ones = jnp.ones((_L,), jnp.float32)

    def compute_piece(buf, off_vec):
        @plsc.parallel_loop(0, _R * _W // _L, unroll=_U)
        def _(vi):
            r = vi >> 5
            c = (vi & 31) * _L
            v = buf[r, pl.ds(c, _L)]
            t = v * 512.0 + 512.0
            t = jnp.minimum(jnp.maximum(t, 0.0), 1023.0)
            b16 = jnp.bitwise_and(t.astype(jnp.int32), (_BINS - 1) * _L)
            plsc.addupdate_scatter(acc, [b16 + off_vec], ones)

    for h, src in ((0, gen_hbm), (1, gen_hbm), (2, tgt_hbm), (3, tgt_hbm)):
        ch = 1 + (h & 1)
        off_vec = lane + h * _BINS * _L

        def piece_src(p, src=src, ch=ch):
            return src.at[w, ch, pl.ds(p * _R, _R), :]

        pltpu.async_copy(piece_src(0), buf0, sem0)

        def pair_body(j, _, piece_src=piece_src, off_vec=off_vec):
            pltpu.async_copy(piece_src(2 * j + 1), buf1, sem1)
            pltpu.make_async_copy(piece_src(0), buf0, sem0).wait()
            compute_piece(buf0, off_vec)

            @pl.when(j < _PIECES // 2 - 1)
            def _():
                pltpu.async_copy(piece_src(2 * j + 2), buf0, sem0)

            pltpu.make_async_copy(piece_src(0), buf1, sem1).wait()
            compute_piece(buf1, off_vec)
            return 0

        lax.fori_loop(0, _PIECES // 2, pair_body, 0)

    pltpu.sync_copy(acc, out_hbm.at[w])


@functools.lru_cache(maxsize=None)
def _build_sc_hist():
    mesh = plsc.VectorSubcoreMesh(
        core_axis_name="c", subcore_axis_name="s",
        num_cores=_NC, num_subcores=_NS)
    return pl.kernel(
        _sc_hist_body,
        out_type=jax.ShapeDtypeStruct((_NW, _ACC), jnp.float32),
        mesh=mesh,
        scratch_types=[
            pltpu.VMEM((_R, _W), jnp.float32),
            pltpu.VMEM((_R, _W), jnp.float32),
            pltpu.VMEM((_ACC,), jnp.float32),
            pltpu.SemaphoreType.DMA,
            pltpu.SemaphoreType.DMA,
        ],
        compiler_params=pltpu.CompilerParams(needs_layout_passes=False),
    )


def _finalize_body(x_ref, o_ref):
    x = x_ref[...]
    s = jnp.sum(x, axis=2)
    s = jnp.sum(s, axis=0, keepdims=True)

    def _norm(hh):
        hh = hh + _EPS
        return hh / jnp.sum(hh)

    g1 = _norm(s[:, 0:_BINS])
    g2 = _norm(s[:, _BINS:2 * _BINS])
    t1 = _norm(s[:, 2 * _BINS:3 * _BINS])
    t2 = _norm(s[:, 3 * _BINS:4 * _BINS])
    kl = (jnp.sum(t1 * (jnp.log(t1) - jnp.log(g1 + _EPS)))
          + jnp.sum(t2 * (jnp.log(t2) - jnp.log(g2 + _EPS))))
    o_ref[0, 0] = kl / (2 * _BINS)


def kernel(generated, target):
    parts = _build_sc_hist()(generated, target)
    parts3 = parts.reshape(_NW, 4 * _BINS, _L)
    out = pl.pallas_call(
        _finalize_body,
        out_shape=jax.ShapeDtypeStruct((1, 1), jnp.float32),
        out_specs=pl.BlockSpec(memory_space=pltpu.SMEM),
    )(parts3)
    return out.reshape(())


# speedup vs baseline: 237.3187x; 1.0054x over previous
import functools

import jax
import jax.numpy as jnp
from jax import lax
from jax.experimental import pallas as pl
from jax.experimental.pallas import tpu as pltpu
from jax.experimental.pallas import tpu_sc as plsc

_BINS = 64
_EPS = 1e-08
_NC, _NS, _L = 2, 16, 16
_NW = _NC * _NS
_W = 512
_R = 64
_PIECES = 512 // _R
_U = 16
_ACC = 4 * _BINS * _L


def _sc_hist_body(gen_hbm, tgt_hbm, out_hbm, buf0, buf1, acc, sem0, sem1):
    w = lax.axis_index("s") * _NC + lax.axis_index("c")

    zeros = jnp.zeros((_L,), jnp.float32)

    def zero_body(i, _):
        acc[pl.ds(i * _L, _L)] = zeros
        return 0

    lax.fori_loop(0, _ACC // _L, zero_body, 0)

    lane = lax.iota(jnp.int32, _L)
    ones = jnp.ones((_L,), jnp.float32)

    def compute_piece(buf, off_vec):
        @plsc.parallel_loop(0, _R * _W // _L, unroll=_U)
        def _(vi):
            r = vi >> 5
            c = (vi & 31) * _L
            v = buf[r, pl.ds(c, _L)]
            t = v * 512.0 + 512.0
            t = jnp.minimum(jnp.maximum(t, 0.0), 1023.0)
            b16 = jnp.bitwise_and(t.astype(jnp.int32), (_BINS - 1) * _L)
            plsc.addupdate_scatter(acc, [b16 + off_vec], ones)

    for h, src in ((0, gen_hbm), (1, gen_hbm), (2, tgt_hbm), (3, tgt_hbm)):
        ch = 1 + (h & 1)
        off_vec = lane + h * _BINS * _L

        def piece_src(p, src=src, ch=ch):
            return src.at[w, ch, pl.ds(p * _R, _R), :]

        pltpu.async_copy(piece_src(0), buf0, sem0)

        def pair_body(j, _, piece_src=piece_src, off_vec=off_vec):
            pltpu.async_copy(piece_src(2 * j + 1), buf1, sem1)
            pltpu.make_async_copy(piece_src(0), buf0, sem0).wait()
            compute_piece(buf0, off_vec)

            @pl.when(j < _PIECES // 2 - 1)
            def _():
                pltpu.async_copy(piece_src(2 * j + 2), buf0, sem0)

            pltpu.make_async_copy(piece_src(0), buf1, sem1).wait()
            compute_piece(buf1, off_vec)
            return 0

        lax.fori_loop(0, _PIECES // 2, pair_body, 0)

    pltpu.sync_copy(acc, out_hbm.at[w])


@functools.lru_cache(maxsize=None)
def _build_sc_hist():
    mesh = plsc.VectorSubcoreMesh(
        core_axis_name="c", subcore_axis_name="s",
        num_cores=_NC, num_subcores=_NS)
    return pl.kernel(
        _sc_hist_body,
        out_type=jax.ShapeDtypeStruct((_NW, _ACC), jnp.float32),
        mesh=mesh,
        scratch_types=[
            pltpu.VMEM((_R, _W), jnp.float32),
            pltpu.VMEM((_R, _W), jnp.float32),
            pltpu.VMEM((_ACC,), jnp.float32),
            pltpu.SemaphoreType.DMA,
            pltpu.SemaphoreType.DMA,
        ],
        compiler_params=pltpu.CompilerParams(needs_layout_passes=False),
    )


def _finalize_body(x_ref, o_ref):
    x = x_ref[...]
    s = jnp.sum(x, axis=2)
    s = jnp.sum(s, axis=0, keepdims=True)

    def _norm(hh):
        hh = hh + _EPS
        return hh / jnp.sum(hh)

    g1 = _norm(s[:, 0:_BINS])
    g2 = _norm(s[:, _BINS:2 * _BINS])
    t1 = _norm(s[:, 2 * _BINS:3 * _BINS])
    t2 = _norm(s[:, 3 * _BINS:4 * _BINS])
    kl = (jnp.sum(t1 * (jnp.log(t1) - jnp.log(g1 + _EPS)))
          + jnp.sum(t2 * (jnp.log(t2) - jnp.log(g2 + _EPS))))
    o_ref[0, 0] = kl / (2 * _BINS)


def kernel(generated, target):
    parts = _build_sc_hist()(generated, target)
    parts3 = parts.reshape(_NW, 4 * _BINS, _L)
    out = pl.pallas_call(
        _finalize_body,
        out_shape=jax.ShapeDtypeStruct((1, 1), jnp.float32),
        out_specs=pl.BlockSpec(memory_space=pltpu.SMEM),
    )(parts3)
    return out.reshape(())


# speedup vs baseline: 268.0098x; 1.1293x over previous
import functools

import jax
import jax.numpy as jnp
from jax import lax
from jax.experimental import pallas as pl
from jax.experimental.pallas import tpu as pltpu
from jax.experimental.pallas import tpu_sc as plsc

_BINS = 64
_EPS = 1e-08
_NC, _NS, _L = 2, 16, 16
_NW = _NC * _NS
_W = 512
_R = 64
_PIECES = 512 // _R
_U = 16
_ACC = 4 * _BINS * _L


def _sc_hist_body(gen_hbm, tgt_hbm, out_hbm, buf0, buf1, acc, sem0, sem1):
    w = lax.axis_index("s") * _NC + lax.axis_index("c")

    zeros = jnp.zeros((_L,), jnp.float32)

    def zero_body(i, _):
        acc[pl.ds(i * _L, _L)] = zeros
        return 0

    lax.fori_loop(0, _ACC // _L, zero_body, 0)

    lane = lax.iota(jnp.int32, _L)
    ones = jnp.ones((_L,), jnp.float32)

    def compute_piece(buf, off_vec):
        @plsc.parallel_loop(0, _R * _W // _L, unroll=_U)
        def _(vi):
            r = vi >> 5
            c = (vi & 31) * _L
            v = buf[r, pl.ds(c, _L)]
            y = v * 0.5 + 1.5
            y = jnp.minimum(jnp.maximum(y, 1.0), 1.9990234375)
            bits = plsc.bitcast(y, jnp.int32)
            b16 = jnp.bitwise_and(
                lax.shift_right_logical(bits, 13), (_BINS - 1) * _L)
            plsc.addupdate_scatter(acc, [b16 + off_vec], ones)

    for h, src in ((0, gen_hbm), (1, gen_hbm), (2, tgt_hbm), (3, tgt_hbm)):
        ch = 1 + (h & 1)
        off_vec = lane + h * _BINS * _L

        def piece_src(p, src=src, ch=ch):
            return src.at[w, ch, pl.ds(p * _R, _R), :]

        pltpu.async_copy(piece_src(0), buf0, sem0)

        def pair_body(j, _, piece_src=piece_src, off_vec=off_vec):
            pltpu.async_copy(piece_src(2 * j + 1), buf1, sem1)
            pltpu.make_async_copy(piece_src(0), buf0, sem0).wait()
            compute_piece(buf0, off_vec)

            @pl.when(j < _PIECES // 2 - 1)
            def _():
                pltpu.async_copy(piece_src(2 * j + 2), buf0, sem0)

            pltpu.make_async_copy(piece_src(0), buf1, sem1).wait()
            compute_piece(buf1, off_vec)
            return 0

        lax.fori_loop(0, _PIECES // 2, pair_body, 0)

    pltpu.sync_copy(acc, out_hbm.at[w])


@functools.lru_cache(maxsize=None)
def _build_sc_hist():
    mesh = plsc.VectorSubcoreMesh(
        core_axis_name="c", subcore_axis_name="s",
        num_cores=_NC, num_subcores=_NS)
    return pl.kernel(
        _sc_hist_body,
        out_type=jax.ShapeDtypeStruct((_NW, _ACC), jnp.float32),
        mesh=mesh,
        scratch_types=[
            pltpu.VMEM((_R, _W), jnp.float32),
            pltpu.VMEM((_R, _W), jnp.float32),
            pltpu.VMEM((_ACC,), jnp.float32),
            pltpu.SemaphoreType.DMA,
            pltpu.SemaphoreType.DMA,
        ],
        compiler_params=pltpu.CompilerParams(needs_layout_passes=False),
    )


def _finalize_body(x_ref, o_ref):
    x = x_ref[...]
    s = jnp.sum(x, axis=2)
    s = jnp.sum(s, axis=0, keepdims=True)

    def _norm(hh):
        hh = hh + _EPS
        return hh / jnp.sum(hh)

    g1 = _norm(s[:, 0:_BINS])
    g2 = _norm(s[:, _BINS:2 * _BINS])
    t1 = _norm(s[:, 2 * _BINS:3 * _BINS])
    t2 = _norm(s[:, 3 * _BINS:4 * _BINS])
    kl = (jnp.sum(t1 * (jnp.log(t1) - jnp.log(g1 + _EPS)))
          + jnp.sum(t2 * (jnp.log(t2) - jnp.log(g2 + _EPS))))
    o_ref[0, 0] = kl / (2 * _BINS)


def kernel(generated, target):
    parts = _build_sc_hist()(generated, target)
    parts3 = parts.reshape(_NW, 4 * _BINS, _L)
    out = pl.pallas_call(
        _finalize_body,
        out_shape=jax.ShapeDtypeStruct((1, 1), jnp.float32),
        out_specs=pl.BlockSpec(memory_space=pltpu.SMEM),
    )(parts3)
    return out.reshape(())


# speedup vs baseline: 284.0924x; 1.0600x over previous
import functools

import jax
import jax.numpy as jnp
from jax import lax
from jax.experimental import pallas as pl
from jax.experimental.pallas import tpu as pltpu
from jax.experimental.pallas import tpu_sc as plsc

_BINS = 64
_EPS = 1e-08
_NC, _NS, _L = 2, 16, 16
_NW = _NC * _NS
_W = 512
_R = 64
_PIECES = 512 // _R
_U = 16
_ACC = 4 * _BINS * _L


def _sc_hist_body(gen_hbm, tgt_hbm, out_hbm, buf0, buf1, acc, accr, sem0, sem1):
    w = lax.axis_index("s") * _NC + lax.axis_index("c")

    zeros = jnp.zeros((_L,), jnp.float32)

    def zero_body(i, _):
        acc[pl.ds(i * _L, _L)] = zeros
        return 0

    lax.fori_loop(0, _ACC // _L, zero_body, 0)

    lane = lax.iota(jnp.int32, _L)
    ones = jnp.ones((_L,), jnp.float32)

    def compute_piece(buf, off_vec):
        @plsc.parallel_loop(0, _R * _W // _L, unroll=_U)
        def _(vi):
            r = vi >> 5
            c = (vi & 31) * _L
            v = buf[r, pl.ds(c, _L)]
            y = v * 0.5 + 1.5
            y = jnp.minimum(jnp.maximum(y, 1.0), 1.9990234375)
            bits = plsc.bitcast(y, jnp.int32)
            b16 = jnp.bitwise_and(
                lax.shift_right_logical(bits, 13), (_BINS - 1) * _L)
            plsc.addupdate_scatter(acc, [b16 + off_vec], ones)

    for h, src in ((0, gen_hbm), (1, gen_hbm), (2, tgt_hbm), (3, tgt_hbm)):
        ch = 1 + (h & 1)
        off_vec = lane + h * _BINS * _L

        def piece_src(p, src=src, ch=ch):
            return src.at[w, ch, pl.ds(p * _R, _R), :]

        pltpu.async_copy(piece_src(0), buf0, sem0)

        def pair_body(j, _, piece_src=piece_src, off_vec=off_vec):
            pltpu.async_copy(piece_src(2 * j + 1), buf1, sem1)
            pltpu.make_async_copy(piece_src(0), buf0, sem0).wait()
            compute_piece(buf0, off_vec)

            @pl.when(j < _PIECES // 2 - 1)
            def _():
                pltpu.async_copy(piece_src(2 * j + 2), buf0, sem0)

            pltpu.make_async_copy(piece_src(0), buf1, sem1).wait()
            compute_piece(buf1, off_vec)
            return 0

        lax.fori_loop(0, _PIECES // 2, pair_body, 0)

    iota16 = lane * _L

    def red_body(g, _):
        base = g * (_L * _L)
        s = jnp.zeros((_L,), jnp.float32)
        for l in range(_L):
            s = s + plsc.load_gather(acc, [base + iota16 + l])
        accr[pl.ds(g * _L, _L)] = s
        return 0

    lax.fori_loop(0, 4 * _BINS // _L, red_body, 0)

    pltpu.sync_copy(accr, out_hbm.at[w])


@functools.lru_cache(maxsize=None)
def _build_sc_hist():
    mesh = plsc.VectorSubcoreMesh(
        core_axis_name="c", subcore_axis_name="s",
        num_cores=_NC, num_subcores=_NS)
    return pl.kernel(
        _sc_hist_body,
        out_type=jax.ShapeDtypeStruct((_NW, 4 * _BINS), jnp.float32),
        mesh=mesh,
        scratch_types=[
            pltpu.VMEM((_R, _W), jnp.float32),
            pltpu.VMEM((_R, _W), jnp.float32),
            pltpu.VMEM((_ACC,), jnp.float32),
            pltpu.VMEM((4 * _BINS,), jnp.float32),
            pltpu.SemaphoreType.DMA,
            pltpu.SemaphoreType.DMA,
        ],
        compiler_params=pltpu.CompilerParams(needs_layout_passes=False),
    )


def _finalize_body(x_ref, o_ref):
    x = x_ref[...]
    s = jnp.sum(x, axis=0, keepdims=True)

    def _norm(hh):
        hh = hh + _EPS
        return hh / jnp.sum(hh)

    g1 = _norm(s[:, 0:_BINS])
    g2 = _norm(s[:, _BINS:2 * _BINS])
    t1 = _norm(s[:, 2 * _BINS:3 * _BINS])
    t2 = _norm(s[:, 3 * _BINS:4 * _BINS])
    kl = (jnp.sum(t1 * (jnp.log(t1) - jnp.log(g1 + _EPS)))
          + jnp.sum(t2 * (jnp.log(t2) - jnp.log(g2 + _EPS))))
    o_ref[0, 0] = kl / (2 * _BINS)


def kernel(generated, target):
    parts = _build_sc_hist()(generated, target)
    out = pl.pallas_call(
        _finalize_body,
        out_shape=jax.ShapeDtypeStruct((1, 1), jnp.float32),
        out_specs=pl.BlockSpec(memory_space=pltpu.SMEM),
    )(parts)
    return out.reshape(())


# speedup vs baseline: 302.2097x; 1.0638x over previous
import functools

import jax
import jax.numpy as jnp
from jax import lax
from jax.experimental import pallas as pl
from jax.experimental.pallas import tpu as pltpu
from jax.experimental.pallas import tpu_sc as plsc

_BINS = 64
_EPS = 1e-08
_NC, _NS, _L = 2, 16, 16
_NW = _NC * _NS
_W = 512
_R = 64
_PIECES = 512 // _R
_U = 16
_ACC = 4 * _BINS * _L


def _sc_hist_body(gen_hbm, tgt_hbm, out_hbm, buf0, buf1, acc, accr, sem0, sem1):
    w = lax.axis_index("s") * _NC + lax.axis_index("c")

    zeros = jnp.zeros((_L,), jnp.float32)

    def zero_body(i, _):
        acc[pl.ds(i * _L, _L)] = zeros
        return 0

    lax.fori_loop(0, _ACC // _L, zero_body, 0)

    lane = lax.iota(jnp.int32, _L)
    ones = jnp.ones((_L,), jnp.float32)

    def compute_piece(buf, off_vec):
        @plsc.parallel_loop(0, _R * _W // _L, unroll=_U)
        def _(vi):
            r = vi >> 5
            c = (vi & 31) * _L
            v = buf[r, pl.ds(c, _L)]
            y = v * 0.5 + 1.5
            y = jnp.minimum(jnp.maximum(y, 1.0), 1.9990234375)
            bits = plsc.bitcast(y, jnp.int32)
            b16 = jnp.bitwise_and(
                lax.shift_right_logical(bits, 13), (_BINS - 1) * _L)
            plsc.addupdate_scatter(acc, [b16 + off_vec], ones)

    npc = 2 * _PIECES

    for src, hbase in ((gen_hbm, 0), (tgt_hbm, 2)):

        def piece_src(p, src=src):
            return src.at[w, 1 + (p >> 3), pl.ds((p & 7) * _R, _R), :]

        def off_for(p, hbase=hbase):
            return lane + (hbase + (p >> 3)) * _BINS * _L

        pltpu.async_copy(piece_src(0), buf0, sem0)

        def pair_body(j, _, piece_src=piece_src, off_for=off_for):
            p0 = 2 * j
            pltpu.async_copy(piece_src(p0 + 1), buf1, sem1)
            pltpu.make_async_copy(piece_src(0), buf0, sem0).wait()
            compute_piece(buf0, off_for(p0))

            @pl.when(j < npc // 2 - 1)
            def _():
                pltpu.async_copy(piece_src(p0 + 2), buf0, sem0)

            pltpu.make_async_copy(piece_src(0), buf1, sem1).wait()
            compute_piece(buf1, off_for(p0 + 1))
            return 0

        lax.fori_loop(0, npc // 2, pair_body, 0)

    iota16 = lane * _L

    def red_body(g, _):
        base = g * (_L * _L)
        s = jnp.zeros((_L,), jnp.float32)
        for l in range(_L):
            s = s + plsc.load_gather(acc, [base + iota16 + l])
        accr[pl.ds(g * _L, _L)] = s
        return 0

    lax.fori_loop(0, 4 * _BINS // _L, red_body, 0)

    pltpu.sync_copy(accr, out_hbm.at[w])


@functools.lru_cache(maxsize=None)
def _build_sc_hist():
    mesh = plsc.VectorSubcoreMesh(
        core_axis_name="c", subcore_axis_name="s",
        num_cores=_NC, num_subcores=_NS)
    return pl.kernel(
        _sc_hist_body,
        out_type=jax.ShapeDtypeStruct((_NW, 4 * _BINS), jnp.float32),
        mesh=mesh,
        scratch_types=[
            pltpu.VMEM((_R, _W), jnp.float32),
            pltpu.VMEM((_R, _W), jnp.float32),
            pltpu.VMEM((_ACC,), jnp.float32),
            pltpu.VMEM((4 * _BINS,), jnp.float32),
            pltpu.SemaphoreType.DMA,
            pltpu.SemaphoreType.DMA,
        ],
        compiler_params=pltpu.CompilerParams(needs_layout_passes=False),
    )


def _finalize_body(x_ref, o_ref):
    x = x_ref[...]
    s = jnp.sum(x, axis=0, keepdims=True)

    def _norm(hh):
        hh = hh + _EPS
        return hh / jnp.sum(hh)

    g1 = _norm(s[:, 0:_BINS])
    g2 = _norm(s[:, _BINS:2 * _BINS])
    t1 = _norm(s[:, 2 * _BINS:3 * _BINS])
    t2 = _norm(s[:, 3 * _BINS:4 * _BINS])
    kl = (jnp.sum(t1 * (jnp.log(t1) - jnp.log(g1 + _EPS)))
          + jnp.sum(t2 * (jnp.log(t2) - jnp.log(g2 + _EPS))))
    o_ref[0, 0] = kl / (2 * _BINS)


def kernel(generated, target):
    parts = _build_sc_hist()(generated, target)
    out = pl.pallas_call(
        _finalize_body,
        out_shape=jax.ShapeDtypeStruct((1, 1), jnp.float32),
        out_specs=pl.BlockSpec(memory_space=pltpu.SMEM),
    )(parts)
    return out.reshape(())


# speedup vs baseline: 312.1225x; 1.0328x over previous
import functools

import jax
import jax.numpy as jnp
from jax import lax
from jax.experimental import pallas as pl
from jax.experimental.pallas import tpu as pltpu
from jax.experimental.pallas import tpu_sc as plsc

_BINS = 64
_EPS = 1e-08
_NC, _NS, _L = 2, 16, 16
_NW = _NC * _NS
_W = 512
_R = 64
_PIECES = 512 // _R
_U = 16
_ACC = 4 * _BINS * _L


def _sc_hist_body(gen_hbm, tgt_hbm, out_hbm, buf0, buf1, acc, accr, sem0, sem1):
    w = lax.axis_index("s") * _NC + lax.axis_index("c")

    zeros = jnp.zeros((_L,), jnp.float32)

    def zero_body(i, _):
        acc[pl.ds(i * _L, _L)] = zeros
        return 0

    lax.fori_loop(0, _ACC // _L, zero_body, 0)

    lane = lax.iota(jnp.int32, _L)
    ones = jnp.ones((_L,), jnp.float32)

    def compute_piece(buf, off_vec):
        @plsc.parallel_loop(0, _R * _W // _L, unroll=_U)
        def _(vi):
            r = vi >> 5
            c = (vi & 31) * _L
            v = buf[r, pl.ds(c, _L)]
            y = v * 0.5 + 1.5
            y = jnp.minimum(jnp.maximum(y, 1.0), 1.9990234375)
            bits = plsc.bitcast(y, jnp.int32)
            b16 = jnp.bitwise_and(
                lax.shift_right_logical(bits, 13), (_BINS - 1) * _L)
            plsc.addupdate_scatter(acc, [b16 + off_vec], ones)

    npc = 2 * _PIECES

    def gen_piece(p):
        return gen_hbm.at[w, 1 + (p >> 3), pl.ds((p & 7) * _R, _R), :]

    def tgt_piece(p):
        return tgt_hbm.at[w, 1 + (p >> 3), pl.ds((p & 7) * _R, _R), :]

    def off_for(p, hbase):
        return lane + (hbase + (p >> 3)) * _BINS * _L

    pltpu.async_copy(gen_piece(0), buf0, sem0)

    def pair_body(j, _):
        pltpu.async_copy(tgt_piece(j), buf1, sem1)
        pltpu.make_async_copy(gen_piece(0), buf0, sem0).wait()
        compute_piece(buf0, off_for(j, 0))

        @pl.when(j < npc - 1)
        def _():
            pltpu.async_copy(gen_piece(j + 1), buf0, sem0)

        pltpu.make_async_copy(tgt_piece(0), buf1, sem1).wait()
        compute_piece(buf1, off_for(j, 2))
        return 0

    lax.fori_loop(0, npc, pair_body, 0)

    iota16 = lane * _L

    def red_body(g, _):
        base = g * (_L * _L)
        s = jnp.zeros((_L,), jnp.float32)
        for l in range(_L):
            s = s + plsc.load_gather(acc, [base + iota16 + l])
        accr[pl.ds(g * _L, _L)] = s
        return 0

    lax.fori_loop(0, 4 * _BINS // _L, red_body, 0)

    pltpu.sync_copy(accr, out_hbm.at[w])


@functools.lru_cache(maxsize=None)
def _build_sc_hist():
    mesh = plsc.VectorSubcoreMesh(
        core_axis_name="c", subcore_axis_name="s",
        num_cores=_NC, num_subcores=_NS)
    return pl.kernel(
        _sc_hist_body,
        out_type=jax.ShapeDtypeStruct((_NW, 4 * _BINS), jnp.float32),
        mesh=mesh,
        scratch_types=[
            pltpu.VMEM((_R, _W), jnp.float32),
            pltpu.VMEM((_R, _W), jnp.float32),
            pltpu.VMEM((_ACC,), jnp.float32),
            pltpu.VMEM((4 * _BINS,), jnp.float32),
            pltpu.SemaphoreType.DMA,
            pltpu.SemaphoreType.DMA,
        ],
        compiler_params=pltpu.CompilerParams(needs_layout_passes=False),
    )


def _finalize_body(x_ref, o_ref):
    x = x_ref[...]
    s = jnp.sum(x, axis=0, keepdims=True)

    def _norm(hh):
        hh = hh + _EPS
        return hh / jnp.sum(hh)

    g1 = _norm(s[:, 0:_BINS])
    g2 = _norm(s[:, _BINS:2 * _BINS])
    t1 = _norm(s[:, 2 * _BINS:3 * _BINS])
    t2 = _norm(s[:, 3 * _BINS:4 * _BINS])
    kl = (jnp.sum(t1 * (jnp.log(t1) - jnp.log(g1 + _EPS)))
          + jnp.sum(t2 * (jnp.log(t2) - jnp.log(g2 + _EPS))))
    o_ref[0, 0] = kl / (2 * _BINS)


def kernel(generated, target):
    parts = _build_sc_hist()(generated, target)
    out = pl.pallas_call(
        _finalize_body,
        out_shape=jax.ShapeDtypeStruct((1, 1), jnp.float32),
        out_specs=pl.BlockSpec(memory_space=pltpu.SMEM),
    )(parts)
    return out.reshape(())
